# Initial kernel scaffold; baseline (speedup 1.0000x reference)
import functools

import jax
import jax.numpy as jnp
from jax import lax
from jax.experimental import pallas as pl
from jax.experimental.pallas import tpu as pltpu
from jax.experimental.pallas import tpu_sc as plsc

EPS = 1e-12

NC = 2
NS = 16
NW = NC * NS


def _sc_gather_rows(prev_flat, ans_emb, ocr_flat, *, A, O, S, P, H):
    PPW = P // NW
    C = 80
    NCH = PPW // C
    VPC = C // 16
    DUMP = P

    mesh = plsc.VectorSubcoreMesh(
        core_axis_name="c", subcore_axis_name="s", num_cores=NC, num_subcores=NS
    )

    @functools.partial(
        pl.kernel,
        mesh=mesh,
        out_type=jax.ShapeDtypeStruct((P + 8, H), jnp.float32),
        scratch_types=[
            pltpu.VMEM((PPW,), jnp.int32),
            pltpu.VMEM((NCH, C), jnp.int32),
            pltpu.VMEM((NCH, C), jnp.int32),
            pltpu.VMEM((NCH, C), jnp.int32),
            pltpu.VMEM((C, H), jnp.float32),
            pltpu.SemaphoreType.DMA,
        ],
    )
    def k(prev_hbm, ans_hbm, ocr_hbm, out_hbm, idx_v, asrc, osrc, odst, buf, sem):
        wid = lax.axis_index("s") * NC + lax.axis_index("c")
        base = wid * PPW
        pltpu.sync_copy(prev_hbm.at[pl.ds(base, PPW)], idx_v)
        for i in range(PPW // 16):
            iv = idx_v[pl.ds(i * 16, 16)]
            pvec = lax.iota(jnp.int32, 16) + (base + i * 16)
            bvec = pvec // S
            isocr = iv >= A
            c, j = i // VPC, i % VPC
            asrc[c, pl.ds(j * 16, 16)] = jnp.where(isocr, 0, iv)
            osrc[c, pl.ds(j * 16, 16)] = jnp.where(isocr, iv - A + bvec * O, 0)
            odst[c, pl.ds(j * 16, 16)] = jnp.where(isocr, pvec, DUMP)
        for c in range(NCH):
            pltpu.async_copy(ans_hbm.at[asrc.at[c]], buf, sem).wait()
            pltpu.sync_copy(buf, out_hbm.at[pl.ds(base + c * C, C)])
            pltpu.async_copy(ocr_hbm.at[osrc.at[c]], buf, sem).wait()
            pltpu.async_copy(buf, out_hbm.at[odst.at[c]], sem).wait()

    return k(prev_flat, ans_emb, ocr_flat)


def _tc_finish(raw_rows, prev3, pos_tiled, type01, params, *, A, P, H, RB):
    def body(raw_ref, prev_ref, pos_ref, type_ref, par_ref, out_ref):
        x = raw_ref[...]
        inds = prev_ref[0, 0, :]
        mask = (inds >= A).reshape(RB, 1)
        m = jnp.mean(x, axis=-1, keepdims=True)
        xc = x - m
        v = jnp.mean(xc * xc, axis=-1, keepdims=True)
        xn = xc * lax.rsqrt(v + EPS)
        g = jnp.where(mask, par_ref[2:3, :], par_ref[0:1, :])
        b = jnp.where(mask, par_ref[3:4, :], par_ref[1:2, :])
        y = xn * g + b
        e = pos_ref[...] + jnp.where(mask, type_ref[1:2, :], type_ref[0:1, :])
        me = jnp.mean(e, axis=-1, keepdims=True)
        ec = e - me
        ve = jnp.mean(ec * ec, axis=-1, keepdims=True)
        en = ec * lax.rsqrt(ve + EPS) * par_ref[4:5, :] + par_ref[5:6, :]
        out_ref[...] = y + en

    nsteps = P // RB
    return pl.pallas_call(
        body,
        grid=(nsteps,),
        in_specs=[
            pl.BlockSpec((RB, H), lambda i: (i, 0)),
            pl.BlockSpec((1, 1, RB), lambda i: (i, 0, 0)),
            pl.BlockSpec((RB, H), lambda i: (0, 0)),
            pl.BlockSpec((8, H), lambda i: (0, 0)),
            pl.BlockSpec((8, H), lambda i: (0, 0)),
        ],
        out_specs=pl.BlockSpec((RB, H), lambda i: (i, 0)),
        out_shape=jax.ShapeDtypeStruct((P, H), jnp.float32),
    )(raw_rows, prev3, pos_tiled, type01, params)


def kernel(ans_emb, ocr_emb, prev_inds, pos_table, type_table,
           ans_g, ans_b, ocr_g, ocr_b, emb_g, emb_b):
    A, H = ans_emb.shape
    B, O, _ = ocr_emb.shape
    _, S = prev_inds.shape
    P = B * S
    RB = 800

    prev_flat = prev_inds.reshape(P)
    ocr_flat = ocr_emb.reshape(B * O, H)

    raw = _sc_gather_rows(prev_flat, ans_emb, ocr_flat, A=A, O=O, S=S, P=P, H=H)

    prev3 = prev_flat.reshape(P // RB, 1, RB)
    pos_tiled = jnp.tile(pos_table, (RB // S, 1))
    type01 = jnp.concatenate(
        [type_table[:2], jnp.zeros((6, H), type_table.dtype)], axis=0)
    params = jnp.concatenate(
        [x.reshape(1, H) for x in (ans_g, ans_b, ocr_g, ocr_b, emb_g, emb_b)]
        + [jnp.zeros((2, H), ans_g.dtype)], axis=0)

    out = _tc_finish(raw, prev3, pos_tiled, type01, params, A=A, P=P, H=H, RB=RB)
    return out.reshape(B, S, H)


# speedup vs baseline: 24.1551x; 24.1551x over previous
import functools

import jax
import jax.numpy as jnp
from jax import lax
from jax.experimental import pallas as pl
from jax.experimental.pallas import tpu as pltpu
from jax.experimental.pallas import tpu_sc as plsc

EPS = 1e-12

NC = 2
NS = 16
NW = NC * NS


def _sc_gather_rows(prevT_flat, ans_emb, ocr_flat, *, A, O, B, P, H):
    PPW = P // NW
    C = 80
    NCH = PPW // C
    VPC = C // 16

    mesh = plsc.VectorSubcoreMesh(
        core_axis_name="c", subcore_axis_name="s", num_cores=NC, num_subcores=NS
    )

    @functools.partial(
        pl.kernel,
        mesh=mesh,
        out_type=(
            jax.ShapeDtypeStruct((2 * P, H), jnp.float32),
            jax.ShapeDtypeStruct((P,), jnp.float32),
        ),
        scratch_types=[
            pltpu.VMEM((PPW,), jnp.int32),
            pltpu.VMEM((NCH, C), jnp.int32),
            pltpu.VMEM((NCH, C), jnp.int32),
            pltpu.VMEM((NCH, C), jnp.int32),
            pltpu.VMEM((PPW,), jnp.float32),
            pltpu.VMEM((C, H), jnp.float32),
            pltpu.VMEM((C, H), jnp.float32),
            pltpu.SemaphoreType.DMA,
            pltpu.SemaphoreType.DMA,
            pltpu.SemaphoreType.DMA,
            pltpu.SemaphoreType.DMA,
        ],
    )
    def k(prev_hbm, ans_hbm, ocr_hbm, out_hbm, mask_hbm,
          idx_v, asrc, osrc, odst, mbuf, buf0, buf1, gs0, gs1, ss0, ss1):
        bufs, gsem, ssem = (buf0, buf1), (gs0, gs1), (ss0, ss1)
        wid = lax.axis_index("s") * NC + lax.axis_index("c")
        base = wid * PPW
        pltpu.sync_copy(prev_hbm.at[pl.ds(base, PPW)], idx_v)
        for i in range(PPW // 16):
            iv = idx_v[pl.ds(i * 16, 16)]
            pvec = lax.iota(jnp.int32, 16) + (base + i * 16)
            bvec = pvec & (B - 1)
            isocr = iv >= A
            c, j = i // VPC, i % VPC
            asrc[c, pl.ds(j * 16, 16)] = jnp.where(isocr, pvec & (2048 - 1), iv)
            osrc[c, pl.ds(j * 16, 16)] = jnp.where(
                isocr, iv - A + bvec * O, pvec & (4096 - 1))
            odst[c, pl.ds(j * 16, 16)] = jnp.where(isocr, pvec, pvec + P)
            mbuf[pl.ds(i * 16, 16)] = jnp.where(
                isocr, jnp.full((16,), 1.0, jnp.float32),
                jnp.full((16,), 0.0, jnp.float32))
        pltpu.sync_copy(mbuf, mask_hbm.at[pl.ds(base, PPW)])

        NT = 2 * NCH

        def start_gather(t, b):
            c = t % NCH
            if t < NCH:
                return pltpu.async_copy(ans_hbm.at[asrc.at[c]], bufs[b], gsem[b])
            return pltpu.async_copy(ocr_hbm.at[osrc.at[c]], bufs[b], gsem[b])

        def start_store(t, b):
            c = t % NCH
            if t < NCH:
                return pltpu.async_copy(
                    bufs[b], out_hbm.at[pl.ds(base + c * C, C)], ssem[b])
            return pltpu.async_copy(bufs[b], out_hbm.at[odst.at[c]], ssem[b])

        gcp = [None] * NT
        scp = [None] * NT
        gcp[0] = start_gather(0, 0)
        for t in range(NT):
            b = t & 1
            gcp[t].wait()
            if t + 1 < NT:
                if t >= 1:
                    scp[t - 1].wait()
                gcp[t + 1] = start_gather(t + 1, (t + 1) & 1)
            scp[t] = start_store(t, b)
        scp[NT - 2].wait()
        scp[NT - 1].wait()

    return k(prevT_flat, ans_emb, ocr_flat)


def _tc_finish(raw3, mask3, pos3, type3, par3, *, S, B, H, SB):
    def body(raw_ref, mask_ref, pos_ref, type_ref, par_ref, out_ref):
        x = raw_ref[...]
        mask = mask_ref[...] != 0.0
        m = jnp.mean(x, axis=-1, keepdims=True)
        xc = x - m
        v = jnp.mean(xc * xc, axis=-1, keepdims=True)
        xn = xc * lax.rsqrt(v + EPS)
        g = jnp.where(mask, par_ref[2:3], par_ref[0:1])
        b = jnp.where(mask, par_ref[3:4], par_ref[1:2])
        y = xn * g + b

        def emb_ln(e):
            me = jnp.mean(e, axis=-1, keepdims=True)
            ec = e - me
            ve = jnp.mean(ec * ec, axis=-1, keepdims=True)
            return ec * lax.rsqrt(ve + EPS) * par_ref[4:5] + par_ref[5:6]

        pos = pos_ref[...]
        en0 = emb_ln(pos + type_ref[0:1])
        en1 = emb_ln(pos + type_ref[1:2])
        out_ref[...] = y + jnp.where(mask, en1, en0)

    return pl.pallas_call(
        body,
        grid=(S // SB,),
        in_specs=[
            pl.BlockSpec((SB, B, H), lambda i: (i, 0, 0)),
            pl.BlockSpec((SB, B, 1), lambda i: (i, 0, 0)),
            pl.BlockSpec((SB, 1, H), lambda i: (i, 0, 0)),
            pl.BlockSpec((8, 1, H), lambda i: (0, 0, 0)),
            pl.BlockSpec((8, 1, H), lambda i: (0, 0, 0)),
        ],
        out_specs=pl.BlockSpec((SB, B, H), lambda i: (i, 0, 0)),
        out_shape=jax.ShapeDtypeStruct((S, B, H), jnp.float32),
    )(raw3, mask3, pos3, type3, par3)


def kernel(ans_emb, ocr_emb, prev_inds, pos_table, type_table,
           ans_g, ans_b, ocr_g, ocr_b, emb_g, emb_b):
    A, H = ans_emb.shape
    B, O, _ = ocr_emb.shape
    _, S = prev_inds.shape
    P = B * S
    SB = 10

    prevT_flat = jnp.swapaxes(prev_inds, 0, 1).reshape(P)
    ocr_flat = ocr_emb.reshape(B * O, H)

    raw, maskf = _sc_gather_rows(prevT_flat, ans_emb, ocr_flat,
                                 A=A, O=O, B=B, P=P, H=H)

    raw3 = raw.reshape(2 * S, B, H)
    mask3 = maskf.reshape(S, B, 1)
    pos3 = pos_table.reshape(S, 1, H)
    type3 = jnp.concatenate(
        [type_table[:2], jnp.zeros((6, H), type_table.dtype)], axis=0
    ).reshape(8, 1, H)
    par3 = jnp.concatenate(
        [x.reshape(1, H) for x in (ans_g, ans_b, ocr_g, ocr_b, emb_g, emb_b)]
        + [jnp.zeros((2, H), ans_g.dtype)], axis=0).reshape(8, 1, H)

    out3 = _tc_finish(raw3, mask3, pos3, type3, par3, S=S, B=B, H=H, SB=SB)
    return jnp.swapaxes(out3, 0, 1)


# speedup vs baseline: 24.3365x; 1.0075x over previous
import functools

import jax
import jax.numpy as jnp
from jax import lax
from jax.experimental import pallas as pl
from jax.experimental.pallas import tpu as pltpu
from jax.experimental.pallas import tpu_sc as plsc

EPS = 1e-12

NC = 2
NS = 16
NW = NC * NS


def _sc_gather_rows(prevT_flat, ans_emb, ocr_flat, *, A, O, B, P, H):
    PPW = P // NW
    C = 80
    NCH = PPW // C
    VPC = C // 16

    mesh = plsc.VectorSubcoreMesh(
        core_axis_name="c", subcore_axis_name="s", num_cores=NC, num_subcores=NS
    )

    @functools.partial(
        pl.kernel,
        mesh=mesh,
        out_type=(
            jax.ShapeDtypeStruct((2 * P, H), jnp.float32),
            jax.ShapeDtypeStruct((P,), jnp.float32),
        ),
        scratch_types=[
            pltpu.VMEM((PPW,), jnp.int32),
            pltpu.VMEM((NCH, C), jnp.int32),
            pltpu.VMEM((NCH, C), jnp.int32),
            pltpu.VMEM((NCH, C), jnp.int32),
            pltpu.VMEM((PPW,), jnp.float32),
            pltpu.VMEM((C, H), jnp.float32),
            pltpu.VMEM((C, H), jnp.float32),
            pltpu.SemaphoreType.DMA,
            pltpu.SemaphoreType.DMA,
            pltpu.SemaphoreType.DMA,
            pltpu.SemaphoreType.DMA,
        ],
    )
    def k(prev_hbm, ans_hbm, ocr_hbm, out_hbm, mask_hbm,
          idx_v, asrc, osrc, odst, mbuf, buf0, buf1, gs0, gs1, ss0, ss1):
        bufs, gsem, ssem = (buf0, buf1), (gs0, gs1), (ss0, ss1)
        wid = lax.axis_index("s") * NC + lax.axis_index("c")
        base = wid * PPW
        pltpu.sync_copy(prev_hbm.at[pl.ds(base, PPW)], idx_v)
        for i in range(PPW // 16):
            iv = idx_v[pl.ds(i * 16, 16)]
            pvec = lax.iota(jnp.int32, 16) + (base + i * 16)
            bvec = pvec & (B - 1)
            isocr = iv >= A
            c, j = i // VPC, i % VPC
            asrc[c, pl.ds(j * 16, 16)] = jnp.where(isocr, pvec & (2048 - 1), iv)
            osrc[c, pl.ds(j * 16, 16)] = jnp.where(
                isocr, iv - A + bvec * O, pvec & (4096 - 1))
            odst[c, pl.ds(j * 16, 16)] = jnp.where(isocr, pvec, pvec + P)
            mbuf[pl.ds(i * 16, 16)] = jnp.where(
                isocr, jnp.full((16,), 1.0, jnp.float32),
                jnp.full((16,), 0.0, jnp.float32))
        pltpu.sync_copy(mbuf, mask_hbm.at[pl.ds(base, PPW)])

        NT = 2 * NCH

        def start_gather(t, b):
            c = t % NCH
            if t < NCH:
                return pltpu.async_copy(ans_hbm.at[asrc.at[c]], bufs[b], gsem[b])
            return pltpu.async_copy(ocr_hbm.at[osrc.at[c]], bufs[b], gsem[b])

        def start_store(t, b):
            c = t % NCH
            if t < NCH:
                return pltpu.async_copy(
                    bufs[b], out_hbm.at[pl.ds(base + c * C, C)], ssem[b])
            return pltpu.async_copy(bufs[b], out_hbm.at[odst.at[c]], ssem[b])

        gcp = [None] * NT
        scp = [None] * NT
        gcp[0] = start_gather(0, 0)
        for t in range(NT):
            b = t & 1
            gcp[t].wait()
            if t + 1 < NT:
                if t >= 1:
                    scp[t - 1].wait()
                gcp[t + 1] = start_gather(t + 1, (t + 1) & 1)
            scp[t] = start_store(t, b)
        scp[NT - 2].wait()
        scp[NT - 1].wait()

    return k(prevT_flat, ans_emb, ocr_flat)


def _tc_finish(raw3, mask3, pos3, type3, par3, *, S, B, H, SB):
    def body(raw_ref, mask_ref, pos_ref, type_ref, par_ref, out_ref):
        x = raw_ref[...]
        mask = mask_ref[...] != 0.0
        m = jnp.mean(x, axis=-1, keepdims=True)
        xc = x - m
        v = jnp.mean(xc * xc, axis=-1, keepdims=True)
        xn = xc * lax.rsqrt(v + EPS)
        g = jnp.where(mask, par_ref[2:3], par_ref[0:1])

        def emb_ln(e):
            me = jnp.mean(e, axis=-1, keepdims=True)
            ec = e - me
            ve = jnp.mean(ec * ec, axis=-1, keepdims=True)
            return ec * lax.rsqrt(ve + EPS) * par_ref[4:5] + par_ref[5:6]

        pos = pos_ref[...]
        add0 = emb_ln(pos + type_ref[0:1]) + par_ref[1:2]
        add1 = emb_ln(pos + type_ref[1:2]) + par_ref[3:4]
        out_ref[...] = xn * g + jnp.where(mask, add1, add0)

    return pl.pallas_call(
        body,
        grid=(S // SB,),
        in_specs=[
            pl.BlockSpec((SB, B, H), lambda i: (i, 0, 0)),
            pl.BlockSpec((SB, B, 1), lambda i: (i, 0, 0)),
            pl.BlockSpec((SB, 1, H), lambda i: (i, 0, 0)),
            pl.BlockSpec((8, 1, H), lambda i: (0, 0, 0)),
            pl.BlockSpec((8, 1, H), lambda i: (0, 0, 0)),
        ],
        out_specs=pl.BlockSpec((SB, B, H), lambda i: (i, 0, 0)),
        out_shape=jax.ShapeDtypeStruct((S, B, H), jnp.float32),
    )(raw3, mask3, pos3, type3, par3)


def kernel(ans_emb, ocr_emb, prev_inds, pos_table, type_table,
           ans_g, ans_b, ocr_g, ocr_b, emb_g, emb_b):
    A, H = ans_emb.shape
    B, O, _ = ocr_emb.shape
    _, S = prev_inds.shape
    P = B * S
    SB = 10

    prevT_flat = jnp.swapaxes(prev_inds, 0, 1).reshape(P)
    ocr_flat = ocr_emb.reshape(B * O, H)

    raw, maskf = _sc_gather_rows(prevT_flat, ans_emb, ocr_flat,
                                 A=A, O=O, B=B, P=P, H=H)

    raw3 = raw.reshape(2 * S, B, H)
    mask3 = maskf.reshape(S, B, 1)
    pos3 = pos_table.reshape(S, 1, H)
    type3 = jnp.concatenate(
        [type_table[:2], jnp.zeros((6, H), type_table.dtype)], axis=0
    ).reshape(8, 1, H)
    par3 = jnp.concatenate(
        [x.reshape(1, H) for x in (ans_g, ans_b, ocr_g, ocr_b, emb_g, emb_b)]
        + [jnp.zeros((2, H), ans_g.dtype)], axis=0).reshape(8, 1, H)

    out3 = _tc_finish(raw3, mask3, pos3, type3, par3, S=S, B=B, H=H, SB=SB)
    return jnp.swapaxes(out3, 0, 1)


# speedup vs baseline: 24.3941x; 1.0024x over previous
import functools

import jax
import jax.numpy as jnp
from jax import lax
from jax.experimental import pallas as pl
from jax.experimental.pallas import tpu as pltpu
from jax.experimental.pallas import tpu_sc as plsc

EPS = 1e-12

NC = 2
NS = 16
NW = NC * NS


def _sc_gather_rows(prevT_flat, ans_emb, ocr_flat, *, A, O, B, P, H):
    PPW = P // NW
    C = 80
    NCH = PPW // C
    VPC = C // 16

    mesh = plsc.VectorSubcoreMesh(
        core_axis_name="c", subcore_axis_name="s", num_cores=NC, num_subcores=NS
    )

    @functools.partial(
        pl.kernel,
        mesh=mesh,
        out_type=(
            jax.ShapeDtypeStruct((2 * P, H), jnp.float32),
            jax.ShapeDtypeStruct((P,), jnp.float32),
        ),
        scratch_types=[
            pltpu.VMEM((PPW,), jnp.int32),
            pltpu.VMEM((NCH, C), jnp.int32),
            pltpu.VMEM((NCH, C), jnp.int32),
            pltpu.VMEM((NCH, C), jnp.int32),
            pltpu.VMEM((PPW,), jnp.float32),
            pltpu.VMEM((C, H), jnp.float32),
            pltpu.VMEM((C, H), jnp.float32),
            pltpu.SemaphoreType.DMA,
            pltpu.SemaphoreType.DMA,
            pltpu.SemaphoreType.DMA,
            pltpu.SemaphoreType.DMA,
        ],
    )
    def k(prev_hbm, ans_hbm, ocr_hbm, out_hbm, mask_hbm,
          idx_v, asrc, osrc, odst, mbuf, buf0, buf1, gs0, gs1, ss0, ss1):
        bufs, gsem, ssem = (buf0, buf1), (gs0, gs1), (ss0, ss1)
        wid = lax.axis_index("s") * NC + lax.axis_index("c")
        base = wid * PPW
        pltpu.sync_copy(prev_hbm.at[pl.ds(base, PPW)], idx_v)
        for i in range(PPW // 16):
            iv = idx_v[pl.ds(i * 16, 16)]
            pvec = lax.iota(jnp.int32, 16) + (base + i * 16)
            bvec = pvec & (B - 1)
            isocr = iv >= A
            c, j = i // VPC, i % VPC
            asrc[c, pl.ds(j * 16, 16)] = jnp.where(isocr, pvec & (2048 - 1), iv)
            osrc[c, pl.ds(j * 16, 16)] = jnp.where(
                isocr, iv - A + bvec * O, pvec & (4096 - 1))
            odst[c, pl.ds(j * 16, 16)] = jnp.where(isocr, pvec, pvec + P)
            mbuf[pl.ds(i * 16, 16)] = jnp.where(
                isocr, jnp.full((16,), 1.0, jnp.float32),
                jnp.full((16,), 0.0, jnp.float32))
        pltpu.sync_copy(mbuf, mask_hbm.at[pl.ds(base, PPW)])

        NT = 2 * NCH

        def start_gather(t, b):
            c = t % NCH
            if t < NCH:
                return pltpu.async_copy(ans_hbm.at[asrc.at[c]], bufs[b], gsem[b])
            return pltpu.async_copy(ocr_hbm.at[osrc.at[c]], bufs[b], gsem[b])

        def start_store(t, b):
            c = t % NCH
            if t < NCH:
                return pltpu.async_copy(
                    bufs[b], out_hbm.at[pl.ds(base + c * C, C)], ssem[b])
            return pltpu.async_copy(bufs[b], out_hbm.at[odst.at[c]], ssem[b])

        gcp = [None] * NT
        scp = [None] * NT
        gcp[0] = start_gather(0, 0)
        for t in range(NT):
            b = t & 1
            gcp[t].wait()
            if t + 1 < NT:
                if t >= 1:
                    scp[t - 1].wait()
                gcp[t + 1] = start_gather(t + 1, (t + 1) & 1)
            scp[t] = start_store(t, b)
        scp[NT - 2].wait()
        scp[NT - 1].wait()

    return k(prevT_flat, ans_emb, ocr_flat)


def _tc_finish(raw3, mask3, pos3, type3, par3, *, S, B, H, SB):
    def body(raw_ref, mask_ref, pos_ref, type_ref, par_ref, out_ref):
        x = raw_ref[...]
        mask = mask_ref[...] != 0.0
        m = jnp.mean(x, axis=-1, keepdims=True)
        xc = x - m
        v = jnp.mean(xc * xc, axis=-1, keepdims=True)
        xn = xc * lax.rsqrt(v + EPS)
        g = jnp.where(mask, par_ref[2:3], par_ref[0:1])

        def emb_ln(e):
            me = jnp.mean(e, axis=-1, keepdims=True)
            ec = e - me
            ve = jnp.mean(ec * ec, axis=-1, keepdims=True)
            return ec * lax.rsqrt(ve + EPS) * par_ref[4:5] + par_ref[5:6]

        pos = pos_ref[...]
        add0 = emb_ln(pos + type_ref[0:1]) + par_ref[1:2]
        add1 = emb_ln(pos + type_ref[1:2]) + par_ref[3:4]
        out_ref[...] = xn * g + jnp.where(mask, add1, add0)

    return pl.pallas_call(
        body,
        grid=(S // SB,),
        in_specs=[
            pl.BlockSpec((SB, B, H), lambda i: (i, 0, 0)),
            pl.BlockSpec((SB, B, 1), lambda i: (i, 0, 0)),
            pl.BlockSpec((SB, 1, H), lambda i: (i, 0, 0)),
            pl.BlockSpec((8, 1, H), lambda i: (0, 0, 0)),
            pl.BlockSpec((8, 1, H), lambda i: (0, 0, 0)),
        ],
        out_specs=pl.BlockSpec((SB, B, H), lambda i: (i, 0, 0)),
        out_shape=jax.ShapeDtypeStruct((S, B, H), jnp.float32),
    )(raw3, mask3, pos3, type3, par3)


def kernel(ans_emb, ocr_emb, prev_inds, pos_table, type_table,
           ans_g, ans_b, ocr_g, ocr_b, emb_g, emb_b):
    A, H = ans_emb.shape
    B, O, _ = ocr_emb.shape
    _, S = prev_inds.shape
    P = B * S
    SB = 20

    prevT_flat = jnp.swapaxes(prev_inds, 0, 1).reshape(P)
    ocr_flat = ocr_emb.reshape(B * O, H)

    raw, maskf = _sc_gather_rows(prevT_flat, ans_emb, ocr_flat,
                                 A=A, O=O, B=B, P=P, H=H)

    raw3 = raw.reshape(2 * S, B, H)
    mask3 = maskf.reshape(S, B, 1)
    pos3 = pos_table.reshape(S, 1, H)
    type3 = jnp.concatenate(
        [type_table[:2], jnp.zeros((6, H), type_table.dtype)], axis=0
    ).reshape(8, 1, H)
    par3 = jnp.concatenate(
        [x.reshape(1, H) for x in (ans_g, ans_b, ocr_g, ocr_b, emb_g, emb_b)]
        + [jnp.zeros((2, H), ans_g.dtype)], axis=0).reshape(8, 1, H)

    out3 = _tc_finish(raw3, mask3, pos3, type3, par3, S=S, B=B, H=H, SB=SB)
    return jnp.swapaxes(out3, 0, 1)
